# Initial kernel scaffold; baseline (speedup 1.0000x reference)
import jax
import jax.numpy as jnp
from jax.experimental import pallas as pl


def kernel(feats, segment_ids, W_aw, b_aw, W_lin, b_lin):
    raise NotImplementedError("write your pallas kernel here")


# speedup vs baseline: 2.0643x; 2.0643x over previous
import functools

import jax
import jax.numpy as jnp
from jax.experimental import pallas as pl
from jax.experimental.pallas import tpu as pltpu

N = 100000
D = 128
B = 1024
OUT = 128
BLK = 512
N_PAD = ((N + BLK - 1) // BLK) * BLK
NBLK = N_PAD // BLK
NEG_INF = float("-inf")


def _body(ids_ref, feats_ref, waw_ref, baw_ref, wlin_ref, blin_ref, out_ref,
          sum_acc, max_acc):
    i = pl.program_id(0)

    @pl.when(i == 0)
    def _init():
        sum_acc[...] = jnp.zeros((B, D), jnp.float32)
        max_acc[...] = jnp.full((B, D), NEG_INF, jnp.float32)

    feats = feats_ref[...]
    ids = ids_ref[0, 0, :]

    gate_logit = jnp.sum(feats * waw_ref[...], axis=1, keepdims=True)
    w = jax.nn.sigmoid(gate_logit + baw_ref[0, 0])
    wf = w * feats

    seg = jax.lax.broadcasted_iota(jnp.int32, (B, 1), 0)
    onehot = (ids[None, :] == seg).astype(jnp.float32)

    sum_acc[...] += jax.lax.dot_general(
        onehot, wf, (((1,), (0,)), ((), ())),
        preferred_element_type=jnp.float32)

    ids_col = ids[:, None]
    x = feats
    s = 1
    while s < BLK:
        x_sh = jnp.concatenate(
            [jnp.full((s, D), NEG_INF, jnp.float32), x[:-s, :]], axis=0)
        ids_sh = jnp.concatenate(
            [jnp.full((s, 1), -1, jnp.int32), ids_col[:-s, :]], axis=0)
        x = jnp.where(ids_col == ids_sh, jnp.maximum(x, x_sh), x)
        s *= 2

    nxt = jnp.concatenate(
        [ids_col[1:, :], jnp.full((1, 1), -1, jnp.int32)], axis=0)
    run_end = (ids_col != nxt).astype(jnp.float32)

    onehot_re = onehot * run_end[:, 0][None, :]
    sel = jax.lax.dot_general(
        onehot_re, jnp.where(run_end > 0, x, 0.0), (((1,), (0,)), ((), ())),
        preferred_element_type=jnp.float32)
    present = jnp.sum(onehot_re, axis=1, keepdims=True)
    max_acc[...] = jnp.where(present > 0,
                             jnp.maximum(max_acc[...], sel), max_acc[...])

    @pl.when(i == NBLK - 1)
    def _fin():
        h = jnp.concatenate([sum_acc[...], max_acc[...]], axis=1)
        out_ref[...] = jax.lax.dot_general(
            h, wlin_ref[...], (((1,), (0,)), ((), ())),
            preferred_element_type=jnp.float32) + blin_ref[...]


@jax.jit
def kernel(feats, segment_ids, W_aw, b_aw, W_lin, b_lin):
    ids = segment_ids.astype(jnp.int32)
    pad = N_PAD - N
    feats_p = jnp.pad(feats, ((0, pad), (0, 0)))
    ids_p = jnp.pad(ids, (0, pad), constant_values=B)
    ids_p = ids_p.reshape(NBLK, 1, BLK)
    waw_row = W_aw.reshape(1, D)
    baw = b_aw.reshape(1, 1)
    blin = b_lin.reshape(1, OUT)

    grid_spec = pltpu.PrefetchScalarGridSpec(
        num_scalar_prefetch=0,
        grid=(NBLK,),
        in_specs=[
            pl.BlockSpec((1, 1, BLK), lambda i: (i, 0, 0)),
            pl.BlockSpec((BLK, D), lambda i: (i, 0)),
            pl.BlockSpec((1, D), lambda i: (0, 0)),
            pl.BlockSpec((1, 1), lambda i: (0, 0)),
            pl.BlockSpec((2 * D, OUT), lambda i: (0, 0)),
            pl.BlockSpec((1, OUT), lambda i: (0, 0)),
        ],
        out_specs=pl.BlockSpec((B, OUT), lambda i: (0, 0)),
        scratch_shapes=[
            pltpu.VMEM((B, D), jnp.float32),
            pltpu.VMEM((B, D), jnp.float32),
        ],
    )
    return pl.pallas_call(
        _body,
        grid_spec=grid_spec,
        out_shape=jax.ShapeDtypeStruct((B, OUT), jnp.float32),
        compiler_params=pltpu.CompilerParams(
            dimension_semantics=("arbitrary",)),
    )(ids_p, feats_p, waw_row, baw, W_lin, blin)
